# baseline (speedup 1.0000x reference)
import jax
import jax.numpy as jnp
from jax.experimental import pallas as pl
from jax.experimental.pallas import tpu as pltpu


def _moe_step(x_ref, se_ref, w1_ref, b1_ref, w2_ref, b2_ref,
              out_ref, rsum_ref, *, n_experts):
    e = pl.program_id(0)
    bf16 = jnp.bfloat16
    x = x_ref[...]
    xb = x.astype(bf16)
    se = se_ref[0].astype(bf16)

    logits = jax.lax.dot_general(
        xb, se, (((1,), (1,)), ((), ())), preferred_element_type=jnp.float32)

    m = jnp.max(logits, axis=0, keepdims=True)
    p = jnp.exp(logits - m)
    dispatch = (p / jnp.sum(p, axis=0, keepdims=True)).astype(bf16)

    c = p * jnp.exp(m)

    slots = jax.lax.dot_general(
        dispatch, xb, (((0,), (0,)), ((), ())), preferred_element_type=jnp.float32)

    h = jax.nn.gelu(
        jnp.dot(slots.astype(bf16), w1_ref[0].astype(bf16),
                preferred_element_type=jnp.float32)
        + b1_ref[0])
    y = jnp.dot(h.astype(bf16), w2_ref[0].astype(bf16),
                preferred_element_type=jnp.float32) + b2_ref[0]

    contrib = jnp.dot(c.astype(bf16), y.astype(bf16),
                      preferred_element_type=jnp.float32)
    csum = jnp.sum(c, axis=1, keepdims=True)

    @pl.when(e == 0)
    def _():
        out_ref[...] = contrib
        rsum_ref[...] = csum

    @pl.when(e > 0)
    def _():
        out_ref[...] += contrib
        rsum_ref[...] += csum

    @pl.when(e == n_experts - 1)
    def _():
        out_ref[...] = out_ref[...] / rsum_ref[...]


def kernel(x, slot_embeds, w1, b1, w2, b2):
    b, n, d = x.shape
    e, s, _ = slot_embeds.shape
    f = w1.shape[-1]
    x2 = x.reshape(n, d)
    b1r = b1.reshape(e, 1, f)
    b2r = b2.reshape(e, 1, d)

    import functools
    out = pl.pallas_call(
        functools.partial(_moe_step, n_experts=e),
        grid=(e,),
        in_specs=[
            pl.BlockSpec((n, d), lambda i: (0, 0)),
            pl.BlockSpec((1, s, d), lambda i: (i, 0, 0)),
            pl.BlockSpec((1, d, f), lambda i: (i, 0, 0)),
            pl.BlockSpec((1, 1, f), lambda i: (i, 0, 0)),
            pl.BlockSpec((1, f, d), lambda i: (i, 0, 0)),
            pl.BlockSpec((1, 1, d), lambda i: (i, 0, 0)),
        ],
        out_specs=pl.BlockSpec((n, d), lambda i: (0, 0)),
        out_shape=jax.ShapeDtypeStruct((n, d), jnp.float32),
        scratch_shapes=[pltpu.VMEM((n, 1), jnp.float32)],
        compiler_params=pltpu.CompilerParams(
            dimension_semantics=("arbitrary",)),
    )(x2, slot_embeds, w1, b1r, w2, b2r)
    return out.reshape(b, n, d)


# speedup vs baseline: 1.0131x; 1.0131x over previous
import jax
import jax.numpy as jnp
from jax.experimental import pallas as pl
from jax.experimental.pallas import tpu as pltpu


def _moe_step(x_ref, se_ref, w1_ref, b1_ref, w2_ref, b2_ref,
              out_ref, rsum_ref, *, n_experts, s_slots):
    e = pl.program_id(0)
    x = x_ref[...]
    se = se_ref[0]

    logits = jax.lax.dot_general(
        x, se, (((1,), (1,)), ((), ())), preferred_element_type=jnp.float32)

    m = jnp.max(logits, axis=0, keepdims=True)
    p = jnp.exp(logits - m)
    colsum = jnp.sum(p, axis=0, keepdims=True)

    ps = jax.lax.dot_general(
        p, x, (((0,), (0,)), ((), ())), preferred_element_type=jnp.float32)
    slots = ps * (1.0 / colsum).reshape(s_slots, 1)

    h = jax.nn.gelu(
        jnp.dot(slots, w1_ref[0], preferred_element_type=jnp.float32)
        + b1_ref[0])
    y = jnp.dot(h, w2_ref[0], preferred_element_type=jnp.float32) + b2_ref[0]

    em = jnp.exp(m)
    em_col = em.reshape(s_slots, 1)
    ys = y * em_col
    contrib = jnp.dot(p, ys, preferred_element_type=jnp.float32)
    csum = jnp.dot(p, em_col, preferred_element_type=jnp.float32)

    @pl.when(e == 0)
    def _():
        out_ref[...] = contrib
        rsum_ref[...] = csum

    @pl.when(e > 0)
    def _():
        out_ref[...] += contrib
        rsum_ref[...] += csum

    @pl.when(e == n_experts - 1)
    def _():
        out_ref[...] = out_ref[...] / rsum_ref[...]


def kernel(x, slot_embeds, w1, b1, w2, b2):
    b, n, d = x.shape
    e, s, _ = slot_embeds.shape
    f = w1.shape[-1]
    x2 = x.reshape(n, d)
    b1r = b1.reshape(e, 1, f)
    b2r = b2.reshape(e, 1, d)

    import functools
    out = pl.pallas_call(
        functools.partial(_moe_step, n_experts=e, s_slots=s),
        grid=(e,),
        in_specs=[
            pl.BlockSpec((n, d), lambda i: (0, 0)),
            pl.BlockSpec((1, s, d), lambda i: (i, 0, 0)),
            pl.BlockSpec((1, d, f), lambda i: (i, 0, 0)),
            pl.BlockSpec((1, 1, f), lambda i: (i, 0, 0)),
            pl.BlockSpec((1, f, d), lambda i: (i, 0, 0)),
            pl.BlockSpec((1, 1, d), lambda i: (i, 0, 0)),
        ],
        out_specs=pl.BlockSpec((n, d), lambda i: (0, 0)),
        out_shape=jax.ShapeDtypeStruct((n, d), jnp.float32),
        scratch_shapes=[pltpu.VMEM((n, 1), jnp.float32)],
        compiler_params=pltpu.CompilerParams(
            dimension_semantics=("arbitrary",)),
    )(x2, slot_embeds, w1, b1r, w2, b2r)
    return out.reshape(b, n, d)
